# baseline (speedup 1.0000x reference)
import functools

import jax
import jax.numpy as jnp
from jax import lax
from jax.experimental import pallas as pl
from jax.experimental.pallas import tpu as pltpu

B = 1
L = 2048
D = 768
H = 12
DH = D // H
DFF = 3072
E = 8
EPS = 1e-9
NEG = -1e30
HI = jax.lax.Precision.HIGHEST

BLK = 128
NB = (2 * L) // BLK
T = NB + E - 1

INTERPRET = False

F32 = jnp.float32
BF16 = jnp.bfloat16
I32 = jnp.int32


def _ln(y, g, b):
    m = jnp.mean(y, axis=-1, keepdims=True)
    v = jnp.mean((y - m) ** 2, axis=-1, keepdims=True)
    return (y - m) / jnp.sqrt(v + 1e-5) * g + b


def _attn_body(xq_ref, xk_ref, wq_ref, bq_ref, wk_ref, bk_ref, wv_ref,
               bv_ref, attn_ref, o_ref, k2_ref, v2_ref):
    qb = pl.program_id(1)

    @pl.when(qb == 0)
    def _():
        xx = xk_ref[...]
        k2_ref[...] = jnp.dot(xx, wk_ref[...], preferred_element_type=F32,
                              precision=HI) + bk_ref[...]
        v2_ref[...] = jnp.dot(xx, wv_ref[...], preferred_element_type=F32,
                              precision=HI) + bv_ref[...]

    q2 = jnp.dot(xq_ref[...], wq_ref[...], preferred_element_type=F32,
                 precision=HI) + bq_ref[...]
    k2 = k2_ref[...]
    v2 = v2_ref[...]
    lane = lax.broadcasted_iota(jnp.int32, (1, 2 * DH), 1)
    for j in range(2):
        mj = (lane // DH == j).astype(F32)
        s = lax.dot_general((q2 * mj).astype(BF16), k2.astype(BF16),
                            (((1,), (1,)), ((), ())),
                            preferred_element_type=F32) * (1.0 / 8.0)
        m = jnp.max(s, axis=-1, keepdims=True)
        p = jnp.exp(s - m)
        ssum = jnp.sum(p, axis=-1, keepdims=True)
        attn_ref[j] = p / ssum
        o_ref[:, j * DH:(j + 1) * DH] = jnp.dot(
            p.astype(BF16), v2[:, j * DH:(j + 1) * DH].astype(BF16),
            preferred_element_type=F32) / ssum


def _postattn_body(x_ref, o_ref, wo_ref, bo_ref, g_ref, b_ref, x1_ref):
    y = x_ref[...] + jnp.dot(o_ref[...].astype(BF16),
                             wo_ref[...].astype(BF16),
                             preferred_element_type=F32) + bo_ref[...]
    x1_ref[...] = _ln(y, g_ref[...], b_ref[...])


def _cumsum_rows(a):
    n = 1
    while n < L:
        shifted = jnp.concatenate(
            [jnp.zeros((n, E), F32), a[:L - n]], axis=0)
        a = a + shifted
        n *= 2
    return a


def _router_body(x1_ref, gw1_ref, gb1_ref, gw2_ref, gb2_ref,
                 posa_ref, posb_ref, wa_ref, wb_ref, cnt_ref):
    hg = jnp.maximum(
        jnp.dot(x1_ref[...], gw1_ref[...], preferred_element_type=F32)
        + gb1_ref[...], 0.0)
    logits = jnp.dot(hg, gw2_ref[...], preferred_element_type=F32) + gb2_ref[...]
    idx = lax.broadcasted_iota(jnp.int32, (L, E), 1)
    m1 = jnp.max(logits, axis=-1, keepdims=True)
    i1 = jnp.min(jnp.where(logits == m1, idx, E), axis=-1, keepdims=True)
    sel1 = idx == i1
    masked = jnp.where(sel1, NEG, logits)
    m2 = jnp.max(masked, axis=-1, keepdims=True)
    i2 = jnp.min(jnp.where(masked == m2, idx, E), axis=-1, keepdims=True)
    sel2 = idx == i2
    mask = sel1 | sel2
    p = jnp.exp(logits - m1)
    probs = p / jnp.sum(p, axis=-1, keepdims=True)
    g = jnp.where(mask, probs, 0.0)
    w = g / (jnp.sum(g, axis=-1, keepdims=True) + EPS)

    oh1 = sel1.astype(F32)
    oh2 = sel2.astype(F32)
    c0 = _cumsum_rows(oh1)
    c1 = _cumsum_rows(oh2)
    tot0 = c0[L - 1:L, :]
    tot1 = c1[L - 1:L, :]
    counts = tot0 + tot1
    te = lax.broadcasted_iota(jnp.int32, (E, E), 0)
    tf = lax.broadcasted_iota(jnp.int32, (E, E), 1)
    tri = (te < tf).astype(F32)
    offs = jnp.dot(counts, tri, preferred_element_type=F32)
    pos0 = jnp.sum(oh1 * (offs + c0 - 1.0), axis=-1, keepdims=True)
    pos1 = jnp.sum(oh2 * (offs + tot0 + c1 - 1.0), axis=-1, keepdims=True)
    w1 = jnp.sum(oh1 * w, axis=-1, keepdims=True)
    w2 = jnp.sum(oh2 * w, axis=-1, keepdims=True)

    first = i1 < i2
    posa_ref[...] = jnp.where(first, pos0, pos1).astype(I32)
    posb_ref[...] = jnp.where(first, pos1, pos0).astype(I32)
    wa_ref[...] = jnp.where(first, w1, w2)
    wb_ref[...] = jnp.where(first, w2, w1)
    cnt_ref[...] = counts.astype(I32)


def _experts_body(tb_ref, te_ref, tl_ref, th_ref, ti_ref,
                  xs_ref, w1_ref, b1_ref, w2_ref, b2_ref, y_ref):
    t = pl.program_id(0)
    h1 = jnp.maximum(
        jnp.dot(xs_ref[...], w1_ref[0], preferred_element_type=F32)
        + b1_ref[0], 0.0)
    yi = jnp.dot(h1, w2_ref[0], preferred_element_type=F32) + b2_ref[0]
    row = tb_ref[t] * BLK + lax.broadcasted_iota(jnp.int32, (BLK, 1), 0)
    rmask = (row >= tl_ref[t]) & (row < th_ref[t])
    val = jnp.where(rmask, yi, 0.0)

    @pl.when(ti_ref[t] == 1)
    def _():
        y_ref[...] = val

    @pl.when(ti_ref[t] == 0)
    def _():
        y_ref[...] += val


def _final_body(nd, x1_ref, w1_ref, b1_ref, w2_ref, b2_ref,
                ya_ref, yb_ref, wa_ref, wb_ref, g_ref, b_ref,
                out_ref, acc_ref):
    d = pl.program_id(0)
    h1 = jnp.maximum(
        jnp.dot(x1_ref[...], w1_ref[...], preferred_element_type=F32)
        + b1_ref[...], 0.0)
    part = jnp.dot(h1, w2_ref[...], preferred_element_type=F32)

    @pl.when(d == 0)
    def _():
        acc_ref[...] = part

    @pl.when(d != 0)
    def _():
        acc_ref[...] += part

    @pl.when(d == nd - 1)
    def _():
        gen = acc_ref[...] + b2_ref[...]
        total = wa_ref[...] * ya_ref[...] + wb_ref[...] * yb_ref[...]
        t32 = total.astype(BF16).astype(F32)
        y = gen + t32 + x1_ref[...]
        out_ref[...] = _ln(y, g_ref[...], b_ref[...])


def kernel(x, Wq, bq, Wk, bk, Wv, bv, Wo, bo, n1g, n1b, gW1, gb1, gW2, gb2,
           eW1, eb1, eW2, eb2, hW1, hb1, hW2, hb2, n2g, n2b):
    xf = x.reshape(L, D)
    r2 = lambda v: v.reshape(1, -1)

    QBLK = 512
    NQB = L // QBLK
    DH2 = 2 * DH
    attn, o = pl.pallas_call(
        _attn_body,
        grid=(H // 2, NQB),
        in_specs=[
            pl.BlockSpec((QBLK, D), lambda hp, qb: (qb, 0)),
            pl.BlockSpec((L, D), lambda hp, qb: (0, 0)),
            pl.BlockSpec((D, DH2), lambda hp, qb: (0, hp)),
            pl.BlockSpec((1, DH2), lambda hp, qb: (0, hp)),
            pl.BlockSpec((D, DH2), lambda hp, qb: (0, hp)),
            pl.BlockSpec((1, DH2), lambda hp, qb: (0, hp)),
            pl.BlockSpec((D, DH2), lambda hp, qb: (0, hp)),
            pl.BlockSpec((1, DH2), lambda hp, qb: (0, hp)),
        ],
        out_specs=[
            pl.BlockSpec((2, QBLK, L), lambda hp, qb: (hp, qb, 0)),
            pl.BlockSpec((QBLK, DH2), lambda hp, qb: (qb, hp)),
        ],
        out_shape=[
            jax.ShapeDtypeStruct((H, L, L), F32),
            jax.ShapeDtypeStruct((L, D), F32),
        ],
        scratch_shapes=[
            pltpu.VMEM((L, DH2), F32),
            pltpu.VMEM((L, DH2), F32),
        ],
        interpret=INTERPRET,
    )(xf, xf, Wq, r2(bq), Wk, r2(bk), Wv, r2(bv))

    x1 = pl.pallas_call(
        _postattn_body,
        out_shape=jax.ShapeDtypeStruct((L, D), F32),
        interpret=INTERPRET,
    )(xf, o, Wo, r2(bo), r2(n1g), r2(n1b))

    posa, posb, wa, wb, counts = pl.pallas_call(
        _router_body,
        out_shape=[
            jax.ShapeDtypeStruct((L, 1), I32),
            jax.ShapeDtypeStruct((L, 1), I32),
            jax.ShapeDtypeStruct((L, 1), F32),
            jax.ShapeDtypeStruct((L, 1), F32),
            jax.ShapeDtypeStruct((1, E), I32),
        ],
        interpret=INTERPRET,
    )(x1, gW1, r2(gb1), gW2, r2(gb2))

    cnt = counts.reshape(E)
    offs = jnp.concatenate([jnp.zeros((1,), I32), jnp.cumsum(cnt)])
    b_all = jnp.repeat(jnp.arange(NB, dtype=I32), E)
    e_all = jnp.tile(jnp.arange(E, dtype=I32), NB)
    lo = jnp.maximum(offs[e_all], b_all * BLK)
    hi = jnp.minimum(offs[e_all + 1], (b_all + 1) * BLK)
    valid = lo < hi
    rank = jnp.cumsum(valid.astype(I32)) - 1
    cb = jnp.cumsum(valid.reshape(NB, E).astype(I32), axis=1).reshape(-1)
    init = valid & (cb == 1)
    slot = jnp.where(valid, rank, T)
    tile_b = jnp.full((T,), NB - 1, I32).at[slot].set(b_all, mode='drop')
    tile_e = jnp.full((T,), E - 1, I32).at[slot].set(e_all, mode='drop')
    tile_lo = jnp.zeros((T,), I32).at[slot].set(lo, mode='drop')
    tile_hi = jnp.zeros((T,), I32).at[slot].set(hi, mode='drop')
    tile_init = jnp.zeros((T,), I32).at[slot].set(init.astype(I32), mode='drop')

    pa = posa.reshape(L)
    pb = posb.reshape(L)
    tok = jnp.arange(L, dtype=I32)
    sort_tok = (jnp.zeros((2 * L,), I32).at[pa].set(tok).at[pb].set(tok))
    xs = jnp.take(x1, sort_tok, axis=0)

    y_sorted = pl.pallas_call(
        _experts_body,
        grid_spec=pltpu.PrefetchScalarGridSpec(
            num_scalar_prefetch=5,
            grid=(T,),
            in_specs=[
                pl.BlockSpec((BLK, D), lambda t, tb, te, tl, th, ti: (tb[t], 0)),
                pl.BlockSpec((1, D, DFF), lambda t, tb, te, tl, th, ti: (te[t], 0, 0)),
                pl.BlockSpec((1, 1, DFF), lambda t, tb, te, tl, th, ti: (te[t], 0, 0)),
                pl.BlockSpec((1, DFF, D), lambda t, tb, te, tl, th, ti: (te[t], 0, 0)),
                pl.BlockSpec((1, 1, D), lambda t, tb, te, tl, th, ti: (te[t], 0, 0)),
            ],
            out_specs=pl.BlockSpec((BLK, D), lambda t, tb, te, tl, th, ti: (tb[t], 0)),
        ),
        out_shape=jax.ShapeDtypeStruct((2 * L, D), F32),
        interpret=INTERPRET,
    )(tile_b, tile_e, tile_lo, tile_hi, tile_init,
      xs, eW1, eb1.reshape(E, 1, DFF), eW2, eb2.reshape(E, 1, D))

    ya = jnp.take(y_sorted, pa, axis=0)
    yb = jnp.take(y_sorted, pb, axis=0)

    ND5 = 4
    FD5 = DFF // ND5
    out = pl.pallas_call(
        functools.partial(_final_body, ND5),
        grid=(ND5,),
        in_specs=[
            pl.BlockSpec((L, D), lambda d: (0, 0)),
            pl.BlockSpec((D, FD5), lambda d: (0, d)),
            pl.BlockSpec((1, FD5), lambda d: (0, d)),
            pl.BlockSpec((FD5, D), lambda d: (d, 0)),
            pl.BlockSpec((1, D), lambda d: (0, 0)),
            pl.BlockSpec((L, D), lambda d: (0, 0)),
            pl.BlockSpec((L, D), lambda d: (0, 0)),
            pl.BlockSpec((L, 1), lambda d: (0, 0)),
            pl.BlockSpec((L, 1), lambda d: (0, 0)),
            pl.BlockSpec((1, D), lambda d: (0, 0)),
            pl.BlockSpec((1, D), lambda d: (0, 0)),
        ],
        out_specs=pl.BlockSpec((L, D), lambda d: (0, 0)),
        out_shape=jax.ShapeDtypeStruct((L, D), F32),
        scratch_shapes=[pltpu.VMEM((L, D), F32)],
        interpret=INTERPRET,
    )(x1, hW1, r2(hb1), hW2, r2(hb2), ya, yb, wa, wb, r2(n2g), r2(n2b))

    return (out.reshape(B, L, D), attn.reshape(B, H, L, L), jnp.float32(0.0))


# speedup vs baseline: 1.0288x; 1.0288x over previous
import functools

import jax
import jax.numpy as jnp
from jax import lax
from jax.experimental import pallas as pl
from jax.experimental.pallas import tpu as pltpu

B = 1
L = 2048
D = 768
H = 12
DH = D // H
DFF = 3072
E = 8
EPS = 1e-9
NEG = -1e30
HI = jax.lax.Precision.HIGHEST

BLK = 512
NB = (2 * L) // BLK
T = NB + E - 1

INTERPRET = False

F32 = jnp.float32
BF16 = jnp.bfloat16
I32 = jnp.int32


def _ln(y, g, b):
    m = jnp.mean(y, axis=-1, keepdims=True)
    v = jnp.mean((y - m) ** 2, axis=-1, keepdims=True)
    return (y - m) / jnp.sqrt(v + 1e-5) * g + b


def _attn_body(xq_ref, xk_ref, wq_ref, bq_ref, wk_ref, bk_ref, wv_ref,
               bv_ref, attn_ref, o_ref, k2_ref, v2_ref):
    qb = pl.program_id(1)

    @pl.when(qb == 0)
    def _():
        xx = xk_ref[...]
        k2_ref[...] = jnp.dot(xx, wk_ref[...], preferred_element_type=F32,
                              precision=HI) + bk_ref[...]
        v2_ref[...] = jnp.dot(xx, wv_ref[...], preferred_element_type=F32,
                              precision=HI) + bv_ref[...]

    q2 = jnp.dot(xq_ref[...], wq_ref[...], preferred_element_type=F32,
                 precision=HI) + bq_ref[...]
    k2 = k2_ref[...]
    v2 = v2_ref[...]
    lane = lax.broadcasted_iota(jnp.int32, (1, 2 * DH), 1)
    for j in range(2):
        mj = (lane // DH == j).astype(F32)
        s = lax.dot_general((q2 * mj).astype(BF16), k2.astype(BF16),
                            (((1,), (1,)), ((), ())),
                            preferred_element_type=F32) * (1.0 / 8.0)
        m = jnp.max(s, axis=-1, keepdims=True)
        p = jnp.exp(s - m)
        ssum = jnp.sum(p, axis=-1, keepdims=True)
        attn_ref[j] = p / ssum
        o_ref[:, j * DH:(j + 1) * DH] = jnp.dot(
            p.astype(BF16), v2[:, j * DH:(j + 1) * DH].astype(BF16),
            preferred_element_type=F32) / ssum


def _postattn_body(x_ref, o_ref, wo_ref, bo_ref, g_ref, b_ref, x1_ref):
    y = x_ref[...] + jnp.dot(o_ref[...].astype(BF16),
                             wo_ref[...].astype(BF16),
                             preferred_element_type=F32) + bo_ref[...]
    x1_ref[...] = _ln(y, g_ref[...], b_ref[...])


def _cumsum_rows(a):
    n = 1
    while n < L:
        shifted = jnp.concatenate(
            [jnp.zeros((n, E), F32), a[:L - n]], axis=0)
        a = a + shifted
        n *= 2
    return a


def _router_body(x1_ref, gw1_ref, gb1_ref, gw2_ref, gb2_ref,
                 posa_ref, posb_ref, wa_ref, wb_ref, cnt_ref):
    hg = jnp.maximum(
        jnp.dot(x1_ref[...], gw1_ref[...], preferred_element_type=F32)
        + gb1_ref[...], 0.0)
    logits = jnp.dot(hg, gw2_ref[...], preferred_element_type=F32) + gb2_ref[...]
    idx = lax.broadcasted_iota(jnp.int32, (L, E), 1)
    m1 = jnp.max(logits, axis=-1, keepdims=True)
    i1 = jnp.min(jnp.where(logits == m1, idx, E), axis=-1, keepdims=True)
    sel1 = idx == i1
    masked = jnp.where(sel1, NEG, logits)
    m2 = jnp.max(masked, axis=-1, keepdims=True)
    i2 = jnp.min(jnp.where(masked == m2, idx, E), axis=-1, keepdims=True)
    sel2 = idx == i2
    mask = sel1 | sel2
    p = jnp.exp(logits - m1)
    probs = p / jnp.sum(p, axis=-1, keepdims=True)
    g = jnp.where(mask, probs, 0.0)
    w = g / (jnp.sum(g, axis=-1, keepdims=True) + EPS)

    oh1 = sel1.astype(F32)
    oh2 = sel2.astype(F32)
    c0 = _cumsum_rows(oh1)
    c1 = _cumsum_rows(oh2)
    tot0 = c0[L - 1:L, :]
    tot1 = c1[L - 1:L, :]
    counts = tot0 + tot1
    te = lax.broadcasted_iota(jnp.int32, (E, E), 0)
    tf = lax.broadcasted_iota(jnp.int32, (E, E), 1)
    tri = (te < tf).astype(F32)
    offs = jnp.dot(counts, tri, preferred_element_type=F32)
    pos0 = jnp.sum(oh1 * (offs + c0 - 1.0), axis=-1, keepdims=True)
    pos1 = jnp.sum(oh2 * (offs + tot0 + c1 - 1.0), axis=-1, keepdims=True)
    w1 = jnp.sum(oh1 * w, axis=-1, keepdims=True)
    w2 = jnp.sum(oh2 * w, axis=-1, keepdims=True)

    first = i1 < i2
    posa_ref[...] = jnp.where(first, pos0, pos1).astype(I32)
    posb_ref[...] = jnp.where(first, pos1, pos0).astype(I32)
    wa_ref[...] = jnp.where(first, w1, w2)
    wb_ref[...] = jnp.where(first, w2, w1)
    cnt_ref[...] = counts.astype(I32)


def _experts_body(tb_ref, te_ref, tl_ref, th_ref, ti_ref,
                  xs_ref, w1_ref, b1_ref, w2_ref, b2_ref, y_ref):
    t = pl.program_id(0)
    h1 = jnp.maximum(
        jnp.dot(xs_ref[...], w1_ref[0], preferred_element_type=F32)
        + b1_ref[0], 0.0)
    yi = jnp.dot(h1, w2_ref[0], preferred_element_type=F32) + b2_ref[0]
    row = tb_ref[t] * BLK + lax.broadcasted_iota(jnp.int32, (BLK, 1), 0)
    rmask = (row >= tl_ref[t]) & (row < th_ref[t])
    val = jnp.where(rmask, yi, 0.0)

    @pl.when(ti_ref[t] == 1)
    def _():
        y_ref[...] = val

    @pl.when(ti_ref[t] == 0)
    def _():
        y_ref[...] += val


def _final_body(nd, x1_ref, w1_ref, b1_ref, w2_ref, b2_ref,
                ya_ref, yb_ref, wa_ref, wb_ref, g_ref, b_ref,
                out_ref, acc_ref):
    d = pl.program_id(0)
    h1 = jnp.maximum(
        jnp.dot(x1_ref[...], w1_ref[...], preferred_element_type=F32)
        + b1_ref[...], 0.0)
    part = jnp.dot(h1, w2_ref[...], preferred_element_type=F32)

    @pl.when(d == 0)
    def _():
        acc_ref[...] = part

    @pl.when(d != 0)
    def _():
        acc_ref[...] += part

    @pl.when(d == nd - 1)
    def _():
        gen = acc_ref[...] + b2_ref[...]
        total = wa_ref[...] * ya_ref[...] + wb_ref[...] * yb_ref[...]
        t32 = total.astype(BF16).astype(F32)
        y = gen + t32 + x1_ref[...]
        out_ref[...] = _ln(y, g_ref[...], b_ref[...])


def kernel(x, Wq, bq, Wk, bk, Wv, bv, Wo, bo, n1g, n1b, gW1, gb1, gW2, gb2,
           eW1, eb1, eW2, eb2, hW1, hb1, hW2, hb2, n2g, n2b):
    xf = x.reshape(L, D)
    r2 = lambda v: v.reshape(1, -1)

    QBLK = 512
    NQB = L // QBLK
    DH2 = 2 * DH
    attn, o = pl.pallas_call(
        _attn_body,
        grid=(H // 2, NQB),
        in_specs=[
            pl.BlockSpec((QBLK, D), lambda hp, qb: (qb, 0)),
            pl.BlockSpec((L, D), lambda hp, qb: (0, 0)),
            pl.BlockSpec((D, DH2), lambda hp, qb: (0, hp)),
            pl.BlockSpec((1, DH2), lambda hp, qb: (0, hp)),
            pl.BlockSpec((D, DH2), lambda hp, qb: (0, hp)),
            pl.BlockSpec((1, DH2), lambda hp, qb: (0, hp)),
            pl.BlockSpec((D, DH2), lambda hp, qb: (0, hp)),
            pl.BlockSpec((1, DH2), lambda hp, qb: (0, hp)),
        ],
        out_specs=[
            pl.BlockSpec((2, QBLK, L), lambda hp, qb: (hp, qb, 0)),
            pl.BlockSpec((QBLK, DH2), lambda hp, qb: (qb, hp)),
        ],
        out_shape=[
            jax.ShapeDtypeStruct((H, L, L), F32),
            jax.ShapeDtypeStruct((L, D), F32),
        ],
        scratch_shapes=[
            pltpu.VMEM((L, DH2), F32),
            pltpu.VMEM((L, DH2), F32),
        ],
        interpret=INTERPRET,
    )(xf, xf, Wq, r2(bq), Wk, r2(bk), Wv, r2(bv))

    x1 = pl.pallas_call(
        _postattn_body,
        out_shape=jax.ShapeDtypeStruct((L, D), F32),
        interpret=INTERPRET,
    )(xf, o, Wo, r2(bo), r2(n1g), r2(n1b))

    posa, posb, wa, wb, counts = pl.pallas_call(
        _router_body,
        out_shape=[
            jax.ShapeDtypeStruct((L, 1), I32),
            jax.ShapeDtypeStruct((L, 1), I32),
            jax.ShapeDtypeStruct((L, 1), F32),
            jax.ShapeDtypeStruct((L, 1), F32),
            jax.ShapeDtypeStruct((1, E), I32),
        ],
        interpret=INTERPRET,
    )(x1, gW1, r2(gb1), gW2, r2(gb2))

    cnt = counts.reshape(E)
    offs = jnp.concatenate([jnp.zeros((1,), I32), jnp.cumsum(cnt)])
    b_all = jnp.repeat(jnp.arange(NB, dtype=I32), E)
    e_all = jnp.tile(jnp.arange(E, dtype=I32), NB)
    lo = jnp.maximum(offs[e_all], b_all * BLK)
    hi = jnp.minimum(offs[e_all + 1], (b_all + 1) * BLK)
    valid = lo < hi
    rank = jnp.cumsum(valid.astype(I32)) - 1
    cb = jnp.cumsum(valid.reshape(NB, E).astype(I32), axis=1).reshape(-1)
    init = valid & (cb == 1)
    slot = jnp.where(valid, rank, T)
    tile_b = jnp.full((T,), NB - 1, I32).at[slot].set(b_all, mode='drop')
    tile_e = jnp.full((T,), E - 1, I32).at[slot].set(e_all, mode='drop')
    tile_lo = jnp.zeros((T,), I32).at[slot].set(lo, mode='drop')
    tile_hi = jnp.zeros((T,), I32).at[slot].set(hi, mode='drop')
    tile_init = jnp.zeros((T,), I32).at[slot].set(init.astype(I32), mode='drop')

    pa = posa.reshape(L)
    pb = posb.reshape(L)
    tok = jnp.arange(L, dtype=I32)
    sort_tok = (jnp.zeros((2 * L,), I32).at[pa].set(tok).at[pb].set(tok))
    xs = jnp.take(x1, sort_tok, axis=0)

    y_sorted = pl.pallas_call(
        _experts_body,
        grid_spec=pltpu.PrefetchScalarGridSpec(
            num_scalar_prefetch=5,
            grid=(T,),
            in_specs=[
                pl.BlockSpec((BLK, D), lambda t, tb, te, tl, th, ti: (tb[t], 0)),
                pl.BlockSpec((1, D, DFF), lambda t, tb, te, tl, th, ti: (te[t], 0, 0)),
                pl.BlockSpec((1, 1, DFF), lambda t, tb, te, tl, th, ti: (te[t], 0, 0)),
                pl.BlockSpec((1, DFF, D), lambda t, tb, te, tl, th, ti: (te[t], 0, 0)),
                pl.BlockSpec((1, 1, D), lambda t, tb, te, tl, th, ti: (te[t], 0, 0)),
            ],
            out_specs=pl.BlockSpec((BLK, D), lambda t, tb, te, tl, th, ti: (tb[t], 0)),
        ),
        out_shape=jax.ShapeDtypeStruct((2 * L, D), F32),
        interpret=INTERPRET,
    )(tile_b, tile_e, tile_lo, tile_hi, tile_init,
      xs, eW1, eb1.reshape(E, 1, DFF), eW2, eb2.reshape(E, 1, D))

    ya = jnp.take(y_sorted, pa, axis=0)
    yb = jnp.take(y_sorted, pb, axis=0)

    ND5 = 4
    FD5 = DFF // ND5
    out = pl.pallas_call(
        functools.partial(_final_body, ND5),
        grid=(ND5,),
        in_specs=[
            pl.BlockSpec((L, D), lambda d: (0, 0)),
            pl.BlockSpec((D, FD5), lambda d: (0, d)),
            pl.BlockSpec((1, FD5), lambda d: (0, d)),
            pl.BlockSpec((FD5, D), lambda d: (d, 0)),
            pl.BlockSpec((1, D), lambda d: (0, 0)),
            pl.BlockSpec((L, D), lambda d: (0, 0)),
            pl.BlockSpec((L, D), lambda d: (0, 0)),
            pl.BlockSpec((L, 1), lambda d: (0, 0)),
            pl.BlockSpec((L, 1), lambda d: (0, 0)),
            pl.BlockSpec((1, D), lambda d: (0, 0)),
            pl.BlockSpec((1, D), lambda d: (0, 0)),
        ],
        out_specs=pl.BlockSpec((L, D), lambda d: (0, 0)),
        out_shape=jax.ShapeDtypeStruct((L, D), F32),
        scratch_shapes=[pltpu.VMEM((L, D), F32)],
        interpret=INTERPRET,
    )(x1, hW1, r2(hb1), hW2, r2(hb2), ya, yb, wa, wb, r2(n2g), r2(n2b))

    return (out.reshape(B, L, D), attn.reshape(B, H, L, L), jnp.float32(0.0))


# speedup vs baseline: 1.0305x; 1.0017x over previous
import functools

import jax
import jax.numpy as jnp
from jax import lax
from jax.experimental import pallas as pl
from jax.experimental.pallas import tpu as pltpu

B = 1
L = 2048
D = 768
H = 12
DH = D // H
DFF = 3072
E = 8
EPS = 1e-9
NEG = -1e30
HI = jax.lax.Precision.HIGHEST

BLK = 512
NB = (2 * L) // BLK
T = NB + E - 1

F32 = jnp.float32
BF16 = jnp.bfloat16
I32 = jnp.int32


def _ln(y, g, b):
    m = jnp.mean(y, axis=-1, keepdims=True)
    v = jnp.mean((y - m) ** 2, axis=-1, keepdims=True)
    return (y - m) / jnp.sqrt(v + 1e-5) * g + b


def _attn_body(xq_ref, xk_ref, wq_ref, bq_ref, wk_ref, bk_ref, wv_ref,
               bv_ref, attn_ref, o_ref, k2_ref, v2_ref):
    qb = pl.program_id(1)

    @pl.when(qb == 0)
    def _():
        xx = xk_ref[...]
        k2_ref[...] = jnp.dot(xx, wk_ref[...], preferred_element_type=F32,
                              precision=HI) + bk_ref[...]
        v2_ref[...] = jnp.dot(xx, wv_ref[...], preferred_element_type=F32,
                              precision=HI) + bv_ref[...]

    q2 = jnp.dot(xq_ref[...], wq_ref[...], preferred_element_type=F32,
                 precision=HI) + bq_ref[...]
    k2 = k2_ref[...]
    v2 = v2_ref[...]
    lane = lax.broadcasted_iota(jnp.int32, (1, 2 * DH), 1)
    for j in range(2):
        mj = (lane // DH == j).astype(F32)
        s = lax.dot_general((q2 * mj).astype(BF16), k2.astype(BF16),
                            (((1,), (1,)), ((), ())),
                            preferred_element_type=F32) * (1.0 / 8.0)
        m = jnp.max(s, axis=-1, keepdims=True)
        p = jnp.exp(s - m)
        ssum = jnp.sum(p, axis=-1, keepdims=True)
        attn_ref[j] = p / ssum
        o_ref[:, j * DH:(j + 1) * DH] = jnp.dot(
            p.astype(BF16), v2[:, j * DH:(j + 1) * DH].astype(BF16),
            preferred_element_type=F32) / ssum


def _postattn_body(x_ref, o_ref, wo_ref, bo_ref, g_ref, b_ref, x1_ref):
    y = x_ref[...] + jnp.dot(o_ref[...].astype(BF16),
                             wo_ref[...].astype(BF16),
                             preferred_element_type=F32) + bo_ref[...]
    x1_ref[...] = _ln(y, g_ref[...], b_ref[...])


def _cumsum_rows(a):
    n = 1
    while n < L:
        shifted = jnp.concatenate(
            [jnp.zeros((n, E), F32), a[:L - n]], axis=0)
        a = a + shifted
        n *= 2
    return a


def _router_body(x1_ref, gw1_ref, gb1_ref, gw2_ref, gb2_ref,
                 posa_ref, posb_ref, wa_ref, wb_ref, cnt_ref):
    hg = jnp.maximum(
        jnp.dot(x1_ref[...], gw1_ref[...], preferred_element_type=F32)
        + gb1_ref[...], 0.0)
    logits = jnp.dot(hg, gw2_ref[...], preferred_element_type=F32) + gb2_ref[...]
    idx = lax.broadcasted_iota(jnp.int32, (L, E), 1)
    m1 = jnp.max(logits, axis=-1, keepdims=True)
    i1 = jnp.min(jnp.where(logits == m1, idx, E), axis=-1, keepdims=True)
    sel1 = idx == i1
    masked = jnp.where(sel1, NEG, logits)
    m2 = jnp.max(masked, axis=-1, keepdims=True)
    i2 = jnp.min(jnp.where(masked == m2, idx, E), axis=-1, keepdims=True)
    sel2 = idx == i2
    mask = sel1 | sel2
    p = jnp.exp(logits - m1)
    probs = p / jnp.sum(p, axis=-1, keepdims=True)
    g = jnp.where(mask, probs, 0.0)
    w = g / (jnp.sum(g, axis=-1, keepdims=True) + EPS)

    oh1 = sel1.astype(F32)
    oh2 = sel2.astype(F32)
    c0 = _cumsum_rows(oh1)
    c1 = _cumsum_rows(oh2)
    tot0 = c0[L - 1:L, :]
    tot1 = c1[L - 1:L, :]
    counts = tot0 + tot1
    te = lax.broadcasted_iota(jnp.int32, (E, E), 0)
    tf = lax.broadcasted_iota(jnp.int32, (E, E), 1)
    tri = (te < tf).astype(F32)
    offs = jnp.dot(counts, tri, preferred_element_type=F32)
    pos0 = jnp.sum(oh1 * (offs + c0 - 1.0), axis=-1, keepdims=True)
    pos1 = jnp.sum(oh2 * (offs + tot0 + c1 - 1.0), axis=-1, keepdims=True)
    w1 = jnp.sum(oh1 * w, axis=-1, keepdims=True)
    w2 = jnp.sum(oh2 * w, axis=-1, keepdims=True)

    first = i1 < i2
    posa_ref[...] = jnp.where(first, pos0, pos1).astype(I32)
    posb_ref[...] = jnp.where(first, pos1, pos0).astype(I32)
    wa_ref[...] = jnp.where(first, w1, w2)
    wb_ref[...] = jnp.where(first, w2, w1)
    cnt_ref[...] = counts.astype(I32)


def _experts_body(tb_ref, te_ref, tl_ref, th_ref, ti_ref,
                  xs_ref, w1_ref, b1_ref, w2_ref, b2_ref, y_ref):
    t = pl.program_id(0)
    h1 = jnp.maximum(
        jnp.dot(xs_ref[...], w1_ref[0], preferred_element_type=F32)
        + b1_ref[0], 0.0)
    yi = jnp.dot(h1, w2_ref[0], preferred_element_type=F32) + b2_ref[0]
    row = tb_ref[t] * BLK + lax.broadcasted_iota(jnp.int32, (BLK, 1), 0)
    rmask = (row >= tl_ref[t]) & (row < th_ref[t])
    val = jnp.where(rmask, yi, 0.0)

    @pl.when(ti_ref[t] == 1)
    def _():
        y_ref[...] = val

    @pl.when(ti_ref[t] == 0)
    def _():
        y_ref[...] += val


def _final_body(nd, x1_ref, w1_ref, b1_ref, w2_ref, b2_ref,
                ya_ref, yb_ref, wa_ref, wb_ref, g_ref, b_ref,
                out_ref, acc_ref):
    d = pl.program_id(0)
    h1 = jnp.maximum(
        jnp.dot(x1_ref[...], w1_ref[...], preferred_element_type=F32)
        + b1_ref[...], 0.0)
    part = jnp.dot(h1, w2_ref[...], preferred_element_type=F32)

    @pl.when(d == 0)
    def _():
        acc_ref[...] = part

    @pl.when(d != 0)
    def _():
        acc_ref[...] += part

    @pl.when(d == nd - 1)
    def _():
        gen = acc_ref[...] + b2_ref[...]
        total = wa_ref[...] * ya_ref[...] + wb_ref[...] * yb_ref[...]
        t32 = total.astype(BF16).astype(F32)
        y = gen + t32 + x1_ref[...]
        out_ref[...] = _ln(y, g_ref[...], b_ref[...])


def kernel(x, Wq, bq, Wk, bk, Wv, bv, Wo, bo, n1g, n1b, gW1, gb1, gW2, gb2,
           eW1, eb1, eW2, eb2, hW1, hb1, hW2, hb2, n2g, n2b):
    xf = x.reshape(L, D)
    r2 = lambda v: v.reshape(1, -1)

    QBLK = 512
    NQB = L // QBLK
    DH2 = 2 * DH
    attn, o = pl.pallas_call(
        _attn_body,
        grid=(H // 2, NQB),
        in_specs=[
            pl.BlockSpec((QBLK, D), lambda hp, qb: (qb, 0)),
            pl.BlockSpec((L, D), lambda hp, qb: (0, 0)),
            pl.BlockSpec((D, DH2), lambda hp, qb: (0, hp)),
            pl.BlockSpec((1, DH2), lambda hp, qb: (0, hp)),
            pl.BlockSpec((D, DH2), lambda hp, qb: (0, hp)),
            pl.BlockSpec((1, DH2), lambda hp, qb: (0, hp)),
            pl.BlockSpec((D, DH2), lambda hp, qb: (0, hp)),
            pl.BlockSpec((1, DH2), lambda hp, qb: (0, hp)),
        ],
        out_specs=[
            pl.BlockSpec((2, QBLK, L), lambda hp, qb: (hp, qb, 0)),
            pl.BlockSpec((QBLK, DH2), lambda hp, qb: (qb, hp)),
        ],
        out_shape=[
            jax.ShapeDtypeStruct((H, L, L), F32),
            jax.ShapeDtypeStruct((L, D), F32),
        ],
        scratch_shapes=[
            pltpu.VMEM((L, DH2), F32),
            pltpu.VMEM((L, DH2), F32),
        ],
    )(xf, xf, Wq, r2(bq), Wk, r2(bk), Wv, r2(bv))

    x1 = pl.pallas_call(
        _postattn_body,
        out_shape=jax.ShapeDtypeStruct((L, D), F32),
    )(xf, o, Wo, r2(bo), r2(n1g), r2(n1b))

    posa, posb, wa, wb, counts = pl.pallas_call(
        _router_body,
        out_shape=[
            jax.ShapeDtypeStruct((L, 1), I32),
            jax.ShapeDtypeStruct((L, 1), I32),
            jax.ShapeDtypeStruct((L, 1), F32),
            jax.ShapeDtypeStruct((L, 1), F32),
            jax.ShapeDtypeStruct((1, E), I32),
        ],
    )(x1, gW1, r2(gb1), gW2, r2(gb2))

    cnt = counts.reshape(E)
    offs = jnp.concatenate([jnp.zeros((1,), I32), jnp.cumsum(cnt)])
    b_all = jnp.repeat(jnp.arange(NB, dtype=I32), E)
    e_all = jnp.tile(jnp.arange(E, dtype=I32), NB)
    lo = jnp.maximum(offs[e_all], b_all * BLK)
    hi = jnp.minimum(offs[e_all + 1], (b_all + 1) * BLK)
    valid = lo < hi
    rank = jnp.cumsum(valid.astype(I32)) - 1
    cb = jnp.cumsum(valid.reshape(NB, E).astype(I32), axis=1).reshape(-1)
    init = valid & (cb == 1)
    slot = jnp.where(valid, rank, T)
    tile_b = jnp.full((T,), NB - 1, I32).at[slot].set(b_all, mode='drop')
    tile_e = jnp.full((T,), E - 1, I32).at[slot].set(e_all, mode='drop')
    tile_lo = jnp.zeros((T,), I32).at[slot].set(lo, mode='drop')
    tile_hi = jnp.zeros((T,), I32).at[slot].set(hi, mode='drop')
    tile_init = jnp.zeros((T,), I32).at[slot].set(init.astype(I32), mode='drop')

    pa = posa.reshape(L)
    pb = posb.reshape(L)
    tok = jnp.arange(L, dtype=I32)
    sort_tok = (jnp.zeros((2 * L,), I32).at[pa].set(tok).at[pb].set(tok))
    xs = jnp.take(x1, sort_tok, axis=0)

    y_sorted = pl.pallas_call(
        _experts_body,
        grid_spec=pltpu.PrefetchScalarGridSpec(
            num_scalar_prefetch=5,
            grid=(T,),
            in_specs=[
                pl.BlockSpec((BLK, D), lambda t, tb, te, tl, th, ti: (tb[t], 0)),
                pl.BlockSpec((1, D, DFF), lambda t, tb, te, tl, th, ti: (te[t], 0, 0)),
                pl.BlockSpec((1, 1, DFF), lambda t, tb, te, tl, th, ti: (te[t], 0, 0)),
                pl.BlockSpec((1, DFF, D), lambda t, tb, te, tl, th, ti: (te[t], 0, 0)),
                pl.BlockSpec((1, 1, D), lambda t, tb, te, tl, th, ti: (te[t], 0, 0)),
            ],
            out_specs=pl.BlockSpec((BLK, D), lambda t, tb, te, tl, th, ti: (tb[t], 0)),
        ),
        out_shape=jax.ShapeDtypeStruct((2 * L, D), F32),
    )(tile_b, tile_e, tile_lo, tile_hi, tile_init,
      xs, eW1, eb1.reshape(E, 1, DFF), eW2, eb2.reshape(E, 1, D))

    ya = jnp.take(y_sorted, pa, axis=0)
    yb = jnp.take(y_sorted, pb, axis=0)

    ND5 = 4
    FD5 = DFF // ND5
    out = pl.pallas_call(
        functools.partial(_final_body, ND5),
        grid=(ND5,),
        in_specs=[
            pl.BlockSpec((L, D), lambda d: (0, 0)),
            pl.BlockSpec((D, FD5), lambda d: (0, d)),
            pl.BlockSpec((1, FD5), lambda d: (0, d)),
            pl.BlockSpec((FD5, D), lambda d: (d, 0)),
            pl.BlockSpec((1, D), lambda d: (0, 0)),
            pl.BlockSpec((L, D), lambda d: (0, 0)),
            pl.BlockSpec((L, D), lambda d: (0, 0)),
            pl.BlockSpec((L, 1), lambda d: (0, 0)),
            pl.BlockSpec((L, 1), lambda d: (0, 0)),
            pl.BlockSpec((1, D), lambda d: (0, 0)),
            pl.BlockSpec((1, D), lambda d: (0, 0)),
        ],
        out_specs=pl.BlockSpec((L, D), lambda d: (0, 0)),
        out_shape=jax.ShapeDtypeStruct((L, D), F32),
        scratch_shapes=[pltpu.VMEM((L, D), F32)],
    )(x1, hW1, r2(hb1), hW2, r2(hb2), ya, yb, wa, wb, r2(n2g), r2(n2b))

    return (out.reshape(B, L, D), attn.reshape(B, H, L, L), jnp.float32(0.0))


# speedup vs baseline: 1.0694x; 1.0377x over previous
import functools

import jax
import jax.numpy as jnp
from jax import lax
from jax.experimental import pallas as pl
from jax.experimental.pallas import tpu as pltpu

B = 1
L = 2048
D = 768
H = 12
DH = D // H
DFF = 3072
E = 8
EPS = 1e-9
NEG = -1e30
HI = jax.lax.Precision.HIGHEST

F32 = jnp.float32
BF16 = jnp.bfloat16


def _ln(y, g, b):
    m = jnp.mean(y, axis=-1, keepdims=True)
    v = jnp.mean((y - m) ** 2, axis=-1, keepdims=True)
    return (y - m) / jnp.sqrt(v + 1e-5) * g + b


def _attn_body(xq_ref, xk_ref, wq_ref, bq_ref, wk_ref, bk_ref, wv_ref,
               bv_ref, attn_ref, o_ref, k2_ref, v2_ref):
    qb = pl.program_id(1)

    @pl.when(qb == 0)
    def _():
        xx = xk_ref[...]
        k2_ref[...] = jnp.dot(xx, wk_ref[...], preferred_element_type=F32,
                              precision=HI) + bk_ref[...]
        v2_ref[...] = jnp.dot(xx, wv_ref[...], preferred_element_type=F32,
                              precision=HI) + bv_ref[...]

    q2 = jnp.dot(xq_ref[...], wq_ref[...], preferred_element_type=F32,
                 precision=HI) + bq_ref[...]
    k2 = k2_ref[...]
    v2 = v2_ref[...]
    lane = lax.broadcasted_iota(jnp.int32, (1, 2 * DH), 1)
    for j in range(2):
        mj = (lane // DH == j).astype(F32)
        s = lax.dot_general((q2 * mj).astype(BF16), k2.astype(BF16),
                            (((1,), (1,)), ((), ())),
                            preferred_element_type=F32) * (1.0 / 8.0)
        m = jnp.max(s, axis=-1, keepdims=True)
        p = jnp.exp(s - m)
        ssum = jnp.sum(p, axis=-1, keepdims=True)
        attn_ref[j] = p / ssum
        o_ref[:, j * DH:(j + 1) * DH] = jnp.dot(
            p.astype(BF16), v2[:, j * DH:(j + 1) * DH].astype(BF16),
            preferred_element_type=F32) / ssum


def _postattn_body(x_ref, o_ref, wo_ref, bo_ref, g_ref, b_ref, x1_ref):
    y = x_ref[...] + jnp.dot(o_ref[...].astype(BF16),
                             wo_ref[...].astype(BF16),
                             preferred_element_type=F32) + bo_ref[...]
    x1_ref[...] = _ln(y, g_ref[...], b_ref[...])


def _router_body(x1_ref, gw1_ref, gb1_ref, gw2_ref, gb2_ref, gated_ref):
    hg = jnp.maximum(
        jnp.dot(x1_ref[...], gw1_ref[...], preferred_element_type=F32)
        + gb1_ref[...], 0.0)
    logits = jnp.dot(hg, gw2_ref[...], preferred_element_type=F32) + gb2_ref[...]
    idx = lax.broadcasted_iota(jnp.int32, (L, E), 1)
    m1 = jnp.max(logits, axis=-1, keepdims=True)
    i1 = jnp.min(jnp.where(logits == m1, idx, E), axis=-1, keepdims=True)
    sel1 = idx == i1
    masked = jnp.where(sel1, NEG, logits)
    m2 = jnp.max(masked, axis=-1, keepdims=True)
    i2 = jnp.min(jnp.where(masked == m2, idx, E), axis=-1, keepdims=True)
    mask = sel1 | (idx == i2)
    p = jnp.exp(logits - m1)
    probs = p / jnp.sum(p, axis=-1, keepdims=True)
    g = jnp.where(mask, probs, 0.0)
    gated_ref[...] = g / (jnp.sum(g, axis=-1, keepdims=True) + EPS)


def _experts_body(nd, x1_ref, w1_ref, b1_ref, w2_ref, b2_ref, gated_ref,
                  tot_ref, acc_ref):
    e = pl.program_id(0)
    d = pl.program_id(1)
    h1 = jnp.maximum(
        jnp.dot(x1_ref[...], w1_ref[0], preferred_element_type=F32)
        + b1_ref[0], 0.0)
    part = jnp.dot(h1, w2_ref[0], preferred_element_type=F32)

    @pl.when(d == 0)
    def _():
        acc_ref[...] = part

    @pl.when(d != 0)
    def _():
        acc_ref[...] += part

    @pl.when(d == nd - 1)
    def _():
        eidx = lax.broadcasted_iota(jnp.int32, (L, E), 1)
        w = jnp.sum(jnp.where(eidx == e, gated_ref[...], 0.0), axis=-1,
                    keepdims=True)
        contrib = (acc_ref[...] + b2_ref[0]) * w

        @pl.when(e == 0)
        def _():
            tot_ref[...] = contrib

        @pl.when(e != 0)
        def _():
            tot_ref[...] += contrib


def _final_body(nd, x1_ref, w1_ref, b1_ref, w2_ref, b2_ref, tot_ref,
                g_ref, b_ref, out_ref, acc_ref):
    d = pl.program_id(0)
    h1 = jnp.maximum(
        jnp.dot(x1_ref[...], w1_ref[...], preferred_element_type=F32)
        + b1_ref[...], 0.0)
    part = jnp.dot(h1, w2_ref[...], preferred_element_type=F32)

    @pl.when(d == 0)
    def _():
        acc_ref[...] = part

    @pl.when(d != 0)
    def _():
        acc_ref[...] += part

    @pl.when(d == nd - 1)
    def _():
        gen = acc_ref[...] + b2_ref[...]
        t32 = tot_ref[...].astype(BF16).astype(F32)
        y = gen + t32 + x1_ref[...]
        out_ref[...] = _ln(y, g_ref[...], b_ref[...])


def kernel(x, Wq, bq, Wk, bk, Wv, bv, Wo, bo, n1g, n1b, gW1, gb1, gW2, gb2,
           eW1, eb1, eW2, eb2, hW1, hb1, hW2, hb2, n2g, n2b):
    xf = x.reshape(L, D)
    r2 = lambda v: v.reshape(1, -1)

    QBLK = 512
    NQB = L // QBLK
    DH2 = 2 * DH
    attn, o = pl.pallas_call(
        _attn_body,
        grid=(H // 2, NQB),
        in_specs=[
            pl.BlockSpec((QBLK, D), lambda hp, qb: (qb, 0)),
            pl.BlockSpec((L, D), lambda hp, qb: (0, 0)),
            pl.BlockSpec((D, DH2), lambda hp, qb: (0, hp)),
            pl.BlockSpec((1, DH2), lambda hp, qb: (0, hp)),
            pl.BlockSpec((D, DH2), lambda hp, qb: (0, hp)),
            pl.BlockSpec((1, DH2), lambda hp, qb: (0, hp)),
            pl.BlockSpec((D, DH2), lambda hp, qb: (0, hp)),
            pl.BlockSpec((1, DH2), lambda hp, qb: (0, hp)),
        ],
        out_specs=[
            pl.BlockSpec((2, QBLK, L), lambda hp, qb: (hp, qb, 0)),
            pl.BlockSpec((QBLK, DH2), lambda hp, qb: (qb, hp)),
        ],
        out_shape=[
            jax.ShapeDtypeStruct((H, L, L), F32),
            jax.ShapeDtypeStruct((L, D), F32),
        ],
        scratch_shapes=[
            pltpu.VMEM((L, DH2), F32),
            pltpu.VMEM((L, DH2), F32),
        ],
    )(xf, xf, Wq, r2(bq), Wk, r2(bk), Wv, r2(bv))

    x1 = pl.pallas_call(
        _postattn_body,
        out_shape=jax.ShapeDtypeStruct((L, D), F32),
    )(xf, o, Wo, r2(bo), r2(n1g), r2(n1b))

    gated = pl.pallas_call(
        _router_body,
        out_shape=jax.ShapeDtypeStruct((L, E), F32),
    )(x1, gW1, r2(gb1), gW2, r2(gb2))

    ND = 2
    FD = DFF // ND
    total = pl.pallas_call(
        functools.partial(_experts_body, ND),
        grid=(E, ND),
        in_specs=[
            pl.BlockSpec((L, D), lambda e, d: (0, 0)),
            pl.BlockSpec((1, D, FD), lambda e, d: (e, 0, d)),
            pl.BlockSpec((1, 1, FD), lambda e, d: (e, 0, d)),
            pl.BlockSpec((1, FD, D), lambda e, d: (e, d, 0)),
            pl.BlockSpec((1, 1, D), lambda e, d: (e, 0, 0)),
            pl.BlockSpec((L, E), lambda e, d: (0, 0)),
        ],
        out_specs=pl.BlockSpec((L, D), lambda e, d: (0, 0)),
        out_shape=jax.ShapeDtypeStruct((L, D), F32),
        scratch_shapes=[pltpu.VMEM((L, D), F32)],
    )(x1, eW1, eb1.reshape(E, 1, DFF), eW2, eb2.reshape(E, 1, D), gated)

    ND5 = 4
    FD5 = DFF // ND5
    out = pl.pallas_call(
        functools.partial(_final_body, ND5),
        grid=(ND5,),
        in_specs=[
            pl.BlockSpec((L, D), lambda d: (0, 0)),
            pl.BlockSpec((D, FD5), lambda d: (0, d)),
            pl.BlockSpec((1, FD5), lambda d: (0, d)),
            pl.BlockSpec((FD5, D), lambda d: (d, 0)),
            pl.BlockSpec((1, D), lambda d: (0, 0)),
            pl.BlockSpec((L, D), lambda d: (0, 0)),
            pl.BlockSpec((1, D), lambda d: (0, 0)),
            pl.BlockSpec((1, D), lambda d: (0, 0)),
        ],
        out_specs=pl.BlockSpec((L, D), lambda d: (0, 0)),
        out_shape=jax.ShapeDtypeStruct((L, D), F32),
        scratch_shapes=[pltpu.VMEM((L, D), F32)],
    )(x1, hW1, r2(hb1), hW2, r2(hb2), total, r2(n2g), r2(n2b))

    return (out.reshape(B, L, D), attn.reshape(B, H, L, L), jnp.float32(0.0))
